# Initial kernel scaffold; baseline (speedup 1.0000x reference)
import jax
import jax.numpy as jnp
from jax.experimental import pallas as pl


def kernel(id_emb, content_emb, collab_emb, params):
    raise NotImplementedError("write your pallas kernel here")


# speedup vs baseline: 1.6055x; 1.6055x over previous
import jax
import jax.numpy as jnp
from jax.experimental import pallas as pl
from jax.experimental.pallas import tpu as pltpu

B, L = 2, 2048
DM, DC, DK = 1024, 768, 64
D = DM + DC + DK
E, H, K = 8, 512, 2
HG = max(D // 2, 128)
T = B * L
TB = 512


def _ln(x, g, b):
    mu = x.mean(-1, keepdims=True)
    v = ((x - mu) ** 2).mean(-1, keepdims=True)
    return (x - mu) * jax.lax.rsqrt(v + 1e-5) * g + b


def _moe_kernel(x_ref, id_ref,
                gW1_ref, gb1_ref, gln_g_ref, gln_b_ref,
                gW2_ref, gb2_ref, gW3_ref, gb3_ref,
                eW1_ref, eb1_ref, eln1_g_ref, eln1_b_ref,
                eW2_ref, eb2_ref, eW3_ref, eb3_ref,
                eln2_g_ref, eln2_b_ref, Wo_ref, bo_ref, alpha_ref,
                out_ref):
    xb = x_ref[...]

    h = jnp.dot(xb, gW1_ref[...], preferred_element_type=jnp.float32)
    h = h + gb1_ref[...]
    h = jax.nn.relu(_ln(h, gln_g_ref[...], gln_b_ref[...]))
    h2 = jnp.dot(h.astype(jnp.bfloat16), gW2_ref[...],
                 preferred_element_type=jnp.float32) + gb2_ref[...]
    h2 = jax.nn.relu(h2)
    logits = jnp.dot(h2.astype(jnp.bfloat16), gW3_ref[...],
                     preferred_element_type=jnp.float32) + gb3_ref[...]

    probs = jax.nn.softmax(logits, axis=-1)
    eidx = jax.lax.broadcasted_iota(jnp.int32, (TB, E), 1)
    i1 = jnp.argmax(probs, axis=-1).astype(jnp.int32)
    p1 = jnp.max(probs, axis=-1)
    masked = jnp.where(eidx == i1[:, None], -jnp.inf, probs)
    i2 = jnp.argmax(masked, axis=-1).astype(jnp.int32)
    p2 = jnp.max(masked, axis=-1)
    denom = p1 + p2 + 1e-8
    w1 = p1 / denom
    w2 = p2 / denom

    fused = jnp.zeros((TB, H // 4), jnp.float32)
    for e in range(E):
        a = jnp.dot(xb, eW1_ref[e], preferred_element_type=jnp.float32)
        a = a + eb1_ref[e][None, :]
        a = jax.nn.relu(_ln(a, eln1_g_ref[e][None, :], eln1_b_ref[e][None, :]))
        b2 = jnp.dot(a.astype(jnp.bfloat16), eW2_ref[e],
                     preferred_element_type=jnp.float32) + eb2_ref[e][None, :]
        b2 = jax.nn.relu(b2)
        o = jnp.dot(b2.astype(jnp.bfloat16), eW3_ref[e],
                    preferred_element_type=jnp.float32) + eb3_ref[e][None, :]
        o = _ln(o, eln2_g_ref[e][None, :], eln2_b_ref[e][None, :])
        sel = jnp.where(i1 == e, w1, 0.0) + jnp.where(i2 == e, w2, 0.0)
        fused = fused + o * sel[:, None]

    proj = jnp.dot(fused.astype(jnp.bfloat16), Wo_ref[...],
                   preferred_element_type=jnp.float32) + bo_ref[...]
    out_ref[...] = id_ref[...] + alpha_ref[0, 0] * proj


def kernel(id_emb, content_emb, collab_emb, params):
    p = params
    x = jnp.concatenate([id_emb, content_emb, collab_emb], axis=-1)
    x = x.reshape(T, D).astype(jnp.bfloat16)
    id_flat = id_emb.reshape(T, DM)

    bf = jnp.bfloat16
    row = lambda a: a.reshape(1, -1)

    def const_spec(shape):
        return pl.BlockSpec(shape, lambda i: (0,) * len(shape))

    operands = [
        x, id_flat,
        p['gW1'].astype(bf), row(p['gb1']), row(p['gln_g']), row(p['gln_b']),
        p['gW2'].astype(bf), row(p['gb2']), p['gW3'].astype(bf), row(p['gb3']),
        p['eW1'].astype(bf), p['eb1'], p['eln1_g'], p['eln1_b'],
        p['eW2'].astype(bf), p['eb2'], p['eW3'].astype(bf), p['eb3'],
        p['eln2_g'], p['eln2_b'], p['Wo'].astype(bf), row(p['bo']),
        p['alpha'].reshape(1, 1),
    ]
    in_specs = [
        pl.BlockSpec((TB, D), lambda i: (i, 0)),
        pl.BlockSpec((TB, DM), lambda i: (i, 0)),
    ] + [const_spec(op.shape) for op in operands[2:]]

    out = pl.pallas_call(
        _moe_kernel,
        grid=(T // TB,),
        in_specs=in_specs,
        out_specs=pl.BlockSpec((TB, DM), lambda i: (i, 0)),
        out_shape=jax.ShapeDtypeStruct((T, DM), jnp.float32),
    )(*operands)
    return out.reshape(B, L, DM)
